# baseline (speedup 1.0000x reference)
import functools

import jax
import jax.numpy as jnp
from jax.experimental import pallas as pl
from jax.experimental.pallas import tpu as pltpu

EPS = 1e-05


def _ln_body(len_ref, x_hbm, w_ref, b_ref, o_hbm, x_buf, o_buf, isems, osems, *,
             bpb, seq, feat, ch, steps_per_core):
    core = pl.program_id(0)
    step = pl.program_id(1)
    gstep = core * steps_per_core + step
    chunks = seq // ch

    def chunk_copy(g, slot, i, c):
        b = g * bpb + i
        return pltpu.make_async_copy(
            x_hbm.at[b, pl.ds(c * ch, ch), :],
            x_buf.at[slot, i, pl.ds(c * ch, ch), :],
            isems.at[slot, i, c],
        )

    def out_copy(g, oslot, i):
        b = g * bpb + i
        return pltpu.make_async_copy(
            o_buf.at[oslot, i],
            o_hbm.at[b],
            osems.at[oslot, i],
        )

    def start_copies(g, slot):
        for i in range(bpb):
            length = len_ref[g * bpb + i]
            for c in range(chunks):
                @pl.when(c * ch < length)
                def _():
                    chunk_copy(g, slot, i, c).start()

    slot = step % 2
    oslot = step % 2

    @pl.when(step == 0)
    def _():
        start_copies(gstep, 0)

    @pl.when(step + 1 < steps_per_core)
    def _():
        start_copies(gstep + 1, (step + 1) % 2)

    w = w_ref[0]
    bias = b_ref[0]
    row_iota = jax.lax.broadcasted_iota(jnp.int32, (seq, 1), 0)
    for i in range(bpb):
        length = len_ref[gstep * bpb + i]
        @pl.when(step >= 2)
        def _():
            out_copy(gstep - 2, oslot, i).wait()
        for c in range(chunks):
            @pl.when(c * ch < length)
            def _():
                chunk_copy(gstep, slot, i, c).wait()
        x = x_buf[slot, i]
        mean = jnp.sum(x, axis=1, keepdims=True) * (1.0 / feat)
        xc = x - mean
        var = jnp.sum(xc * xc, axis=1, keepdims=True) * (1.0 / feat)
        inv = 1.0 / (jnp.sqrt(var) + EPS)
        y = xc * inv * w + bias
        o_buf[oslot, i] = jnp.where(row_iota < length, y, 0.0)
        out_copy(gstep, oslot, i).start()

    @pl.when(step == steps_per_core - 1)
    def _():
        for i in range(bpb):
            out_copy(gstep - 1, (step - 1) % 2, i).wait()
            out_copy(gstep, oslot, i).wait()


def kernel(x, weights, biases, lengths):
    B, S, F = x.shape
    bpb = 4
    ch = 32
    cores = 2
    steps_per_core = B // bpb // cores
    lengths32 = lengths.astype(jnp.int32)
    w2 = weights.reshape(1, F)
    b2 = biases.reshape(1, F)

    grid_spec = pltpu.PrefetchScalarGridSpec(
        num_scalar_prefetch=1,
        grid=(cores, steps_per_core),
        in_specs=[
            pl.BlockSpec(memory_space=pl.ANY),
            pl.BlockSpec((1, F), lambda co, s, len_ref: (0, 0)),
            pl.BlockSpec((1, F), lambda co, s, len_ref: (0, 0)),
        ],
        out_specs=pl.BlockSpec(memory_space=pl.ANY),
        scratch_shapes=[
            pltpu.VMEM((2, bpb, S, F), jnp.float32),
            pltpu.VMEM((2, bpb, S, F), jnp.float32),
            pltpu.SemaphoreType.DMA((2, bpb, S // ch)),
            pltpu.SemaphoreType.DMA((2, bpb)),
        ],
    )
    return pl.pallas_call(
        functools.partial(_ln_body, bpb=bpb, seq=S, feat=F, ch=ch,
                          steps_per_core=steps_per_core),
        grid_spec=grid_spec,
        out_shape=jax.ShapeDtypeStruct((B, S, F), x.dtype),
        compiler_params=pltpu.CompilerParams(
            dimension_semantics=("parallel", "arbitrary"),
        ),
    )(lengths32, x, w2, b2)


# speedup vs baseline: 1.0119x; 1.0119x over previous
import functools

import jax
import jax.numpy as jnp
from jax.experimental import pallas as pl
from jax.experimental.pallas import tpu as pltpu

EPS = 1e-05


def _ln_body(len_ref, x_hbm, w_ref, b_ref, o_hbm, x_buf, o_buf, isems, osems, *,
             bpb, seq, feat, ch, steps_per_core):
    core = pl.program_id(0)
    step = pl.program_id(1)
    gstep = core * steps_per_core + step
    chunks = seq // ch

    def chunk_copy(g, slot, i, c):
        b = g * bpb + i
        return pltpu.make_async_copy(
            x_hbm.at[b, pl.ds(c * ch, ch), :],
            x_buf.at[slot, i, pl.ds(c * ch, ch), :],
            isems.at[slot, i, c],
        )

    def out_copy(g, oslot, i):
        b = g * bpb + i
        return pltpu.make_async_copy(
            o_buf.at[oslot, i],
            o_hbm.at[b],
            osems.at[oslot, i],
        )

    def start_copies(g, slot):
        for i in range(bpb):
            length = len_ref[g * bpb + i]
            for c in range(chunks):
                @pl.when(c * ch < length)
                def _():
                    chunk_copy(g, slot, i, c).start()

    slot = step % 2
    oslot = step % 2

    @pl.when(step == 0)
    def _():
        start_copies(gstep, 0)

    @pl.when(step + 1 < steps_per_core)
    def _():
        start_copies(gstep + 1, (step + 1) % 2)

    w = w_ref[0]
    bias = b_ref[0]
    row_iota = jax.lax.broadcasted_iota(jnp.int32, (seq, 1), 0)
    for i in range(bpb):
        length = len_ref[gstep * bpb + i]
        @pl.when(step >= 2)
        def _():
            out_copy(gstep - 2, oslot, i).wait()
        for c in range(chunks):
            @pl.when(c * ch < length)
            def _():
                chunk_copy(gstep, slot, i, c).wait()
        x = x_buf[slot, i]
        mean = jnp.sum(x, axis=1, keepdims=True) * (1.0 / feat)
        xc = x - mean
        var = jnp.sum(xc * xc, axis=1, keepdims=True) * (1.0 / feat)
        inv = 1.0 / (jnp.sqrt(var) + EPS)
        y = xc * inv * w + bias
        o_buf[oslot, i] = jnp.where(row_iota < length, y, 0.0)
        out_copy(gstep, oslot, i).start()

    @pl.when(step == steps_per_core - 1)
    def _():
        for i in range(bpb):
            out_copy(gstep - 1, (step - 1) % 2, i).wait()
            out_copy(gstep, oslot, i).wait()


def kernel(x, weights, biases, lengths):
    B, S, F = x.shape
    bpb = 4
    ch = 64
    cores = 2
    steps_per_core = B // bpb // cores
    lengths32 = lengths.astype(jnp.int32)
    w2 = weights.reshape(1, F)
    b2 = biases.reshape(1, F)

    grid_spec = pltpu.PrefetchScalarGridSpec(
        num_scalar_prefetch=1,
        grid=(cores, steps_per_core),
        in_specs=[
            pl.BlockSpec(memory_space=pl.ANY),
            pl.BlockSpec((1, F), lambda co, s, len_ref: (0, 0)),
            pl.BlockSpec((1, F), lambda co, s, len_ref: (0, 0)),
        ],
        out_specs=pl.BlockSpec(memory_space=pl.ANY),
        scratch_shapes=[
            pltpu.VMEM((2, bpb, S, F), jnp.float32),
            pltpu.VMEM((2, bpb, S, F), jnp.float32),
            pltpu.SemaphoreType.DMA((2, bpb, S // ch)),
            pltpu.SemaphoreType.DMA((2, bpb)),
        ],
    )
    return pl.pallas_call(
        functools.partial(_ln_body, bpb=bpb, seq=S, feat=F, ch=ch,
                          steps_per_core=steps_per_core),
        grid_spec=grid_spec,
        out_shape=jax.ShapeDtypeStruct((B, S, F), x.dtype),
        compiler_params=pltpu.CompilerParams(
            dimension_semantics=("parallel", "arbitrary"),
        ),
    )(lengths32, x, w2, b2)


# speedup vs baseline: 1.0257x; 1.0136x over previous
import functools

import jax
import jax.numpy as jnp
from jax.experimental import pallas as pl
from jax.experimental.pallas import tpu as pltpu

EPS = 1e-05
TAIL = 32


def _ln_body(len_ref, x_hbm, w_ref, b_ref, o_hbm, x_buf, o_buf, isems, tsems,
             osems, *, bpb, seq, feat, ch, steps_per_core):
    core = pl.program_id(0)
    step = pl.program_id(1)
    gstep = core * steps_per_core + step
    chunks = seq // ch

    def big_copy(g, slot, i, c):
        b = g * bpb + i
        return pltpu.make_async_copy(
            x_hbm.at[b, pl.ds(c * ch, ch), :],
            x_buf.at[slot, i, pl.ds(c * ch, ch), :],
            isems.at[slot, i, c],
        )

    def tail_copy(g, slot, i, k, nbig):
        b = g * bpb + i
        off = nbig * ch + k * TAIL
        return pltpu.make_async_copy(
            x_hbm.at[b, pl.ds(off, TAIL), :],
            x_buf.at[slot, i, pl.ds(off, TAIL), :],
            tsems.at[slot, i, k],
        )

    def out_copy(g, oslot, i):
        b = g * bpb + i
        return pltpu.make_async_copy(
            o_buf.at[oslot, i],
            o_hbm.at[b],
            osems.at[oslot, i],
        )

    def for_each_input_copy(g, slot, fn):
        for i in range(bpb):
            length = len_ref[g * bpb + i]
            rup = (length + TAIL - 1) // TAIL * TAIL
            nbig = rup // ch
            for c in range(chunks):
                @pl.when((c + 1) * ch <= rup)
                def _():
                    fn(big_copy(g, slot, i, c))
            for k in range(ch // TAIL):
                @pl.when(nbig * ch + k * TAIL < rup)
                def _():
                    fn(tail_copy(g, slot, i, k, nbig))

    slot = step % 2
    oslot = step % 2

    @pl.when(step == 0)
    def _():
        for_each_input_copy(gstep, 0, lambda cp: cp.start())

    @pl.when(step + 1 < steps_per_core)
    def _():
        for_each_input_copy(gstep + 1, (step + 1) % 2, lambda cp: cp.start())

    w = w_ref[0]
    bias = b_ref[0]
    row_iota = jax.lax.broadcasted_iota(jnp.int32, (seq, 1), 0)
    for i in range(bpb):
        length = len_ref[gstep * bpb + i]
        rup = (length + TAIL - 1) // TAIL * TAIL
        nbig = rup // ch
        @pl.when(step >= 2)
        def _():
            out_copy(gstep - 2, oslot, i).wait()
        for c in range(chunks):
            @pl.when((c + 1) * ch <= rup)
            def _():
                big_copy(gstep, slot, i, c).wait()
        for k in range(ch // TAIL):
            @pl.when(nbig * ch + k * TAIL < rup)
            def _():
                tail_copy(gstep, slot, i, k, nbig).wait()
        x = x_buf[slot, i]
        mean = jnp.sum(x, axis=1, keepdims=True) * (1.0 / feat)
        xc = x - mean
        var = jnp.sum(xc * xc, axis=1, keepdims=True) * (1.0 / feat)
        inv = 1.0 / (jnp.sqrt(var) + EPS)
        y = xc * inv * w + bias
        o_buf[oslot, i] = jnp.where(row_iota < length, y, 0.0)
        out_copy(gstep, oslot, i).start()

    @pl.when(step == steps_per_core - 1)
    def _():
        for i in range(bpb):
            out_copy(gstep - 1, (step - 1) % 2, i).wait()
            out_copy(gstep, oslot, i).wait()


def kernel(x, weights, biases, lengths):
    B, S, F = x.shape
    bpb = 4
    ch = 256
    cores = 2
    steps_per_core = B // bpb // cores
    lengths32 = lengths.astype(jnp.int32)
    w2 = weights.reshape(1, F)
    b2 = biases.reshape(1, F)

    grid_spec = pltpu.PrefetchScalarGridSpec(
        num_scalar_prefetch=1,
        grid=(cores, steps_per_core),
        in_specs=[
            pl.BlockSpec(memory_space=pl.ANY),
            pl.BlockSpec((1, F), lambda co, s, len_ref: (0, 0)),
            pl.BlockSpec((1, F), lambda co, s, len_ref: (0, 0)),
        ],
        out_specs=pl.BlockSpec(memory_space=pl.ANY),
        scratch_shapes=[
            pltpu.VMEM((2, bpb, S, F), jnp.float32),
            pltpu.VMEM((2, bpb, S, F), jnp.float32),
            pltpu.SemaphoreType.DMA((2, bpb, S // ch)),
            pltpu.SemaphoreType.DMA((2, bpb, ch // TAIL)),
            pltpu.SemaphoreType.DMA((2, bpb)),
        ],
    )
    return pl.pallas_call(
        functools.partial(_ln_body, bpb=bpb, seq=S, feat=F, ch=ch,
                          steps_per_core=steps_per_core),
        grid_spec=grid_spec,
        out_shape=jax.ShapeDtypeStruct((B, S, F), x.dtype),
        compiler_params=pltpu.CompilerParams(
            dimension_semantics=("parallel", "arbitrary"),
        ),
    )(lengths32, x, w2, b2)
